# baseline (speedup 1.0000x reference)
import jax
import jax.numpy as jnp
from jax import lax
from jax.experimental import pallas as pl
from jax.experimental.pallas import tpu as pltpu
from jax.experimental.pallas import tpu_sc as plsc

N_POINTS = 500000
C = 48
RES = 300
F = 27
NC, NS, L = 2, 16, 16
NW = NC * NS
K = 48
N_CHUNKS = 326
N_PAD = NW * K * N_CHUNKS
PER_W = N_PAD // NW
N_PAIRS = N_CHUNKS // 2
GROUPS = K // L
PROD = 3 * C
PRODP = 160


def _coord(v):
    t = (v + 1.0) * 0.5 * (RES - 1)
    fi = jnp.clip(t.astype(jnp.int32), 0, RES - 2)
    w1 = t - fi.astype(jnp.float32)
    return fi, 1.0 - w1, w1


def _sc_body(xs, ys, zs, ptab, ltab_hbm, prod_out,
             xv0, yv0, zv0, xv1, yv1, zv1, ltab_v,
             idx0, idx1, wb0, wb1, lb0, lb1, tap0, tap1, pr0, pr1,
             gsem0, gsem1, xsem0, xsem1, psem0, psem1):
    xv = (xv0, xv1)
    yv = (yv0, yv1)
    zv = (zv0, zv1)
    idxb = (idx0, idx1)
    wbuf = (wb0, wb1)
    lbuf = (lb0, lb1)
    tap = (tap0, tap1)
    prodb = (pr0, pr1)
    gsem = (gsem0, gsem1)
    xsem = (xsem0, xsem1)
    psem = (psem0, psem1)

    wid = lax.axis_index("s") * NC + lax.axis_index("c")
    wbase = wid * PER_W
    iota16 = lax.iota(jnp.int32, L)

    pltpu.sync_copy(ltab_hbm, ltab_v)

    def pass1(cn, b):
        base = wbase + cn * K
        for g in range(GROUPS):
            sl = pl.ds(g * L, L)
            xi, xw0, xw1 = _coord(xv[b][sl])
            yi, yw0, yw1 = _coord(yv[b][sl])
            zi, zw0, zw1 = _coord(zv[b][sl])
            r0 = yi * RES + zi
            r1 = xi * RES + zi + RES * RES
            r2 = xi * RES + yi + 2 * RES * RES
            for t, r in enumerate((r0, r1, r2)):
                idxb[b][4 * t + 0, sl] = r
                idxb[b][4 * t + 1, sl] = r + 1
                idxb[b][4 * t + 2, sl] = r + RES
                idxb[b][4 * t + 3, sl] = r + RES + 1
            wbuf[b][0, sl] = xw0
            wbuf[b][1, sl] = xw1
            wbuf[b][2, sl] = yw0
            wbuf[b][3, sl] = yw1
            wbuf[b][4, sl] = zw0
            wbuf[b][5, sl] = zw1
            lbuf[b][0, sl] = xi * C
            lbuf[b][1, sl] = yi * C + RES * C
            lbuf[b][2, sl] = zi * C + 2 * RES * C

    def issue_gathers(b):
        for t in range(12):
            pltpu.async_copy(ptab.at[idxb[b].at[t]],
                             tap[b].at[pl.ds(t * K, K)], gsem[b])

    def wait_gathers(b):
        pltpu.make_async_copy(ptab.at[pl.ds(0, 12 * K)], tap[b],
                              gsem[b]).wait()

    def issue_xyz(c, b):
        base = wbase + c * K
        pltpu.async_copy(xs.at[pl.ds(base, K)], xv[b], xsem[b])
        pltpu.async_copy(ys.at[pl.ds(base, K)], yv[b], xsem[b])
        pltpu.async_copy(zs.at[pl.ds(base, K)], zv[b], xsem[b])

    def wait_xyz(b):
        for r in (xv[b], yv[b], zv[b]):
            pltpu.make_async_copy(xs.at[pl.ds(0, K)], r, xsem[b]).wait()

    def combine(c, b):
        for g in range(GROUPS):
            sl = pl.ds(g * L, L)
            bw = [wbuf[b][j, sl] for j in range(6)]
            lb = [lbuf[b][j, sl] for j in range(3)]
            ccoff = [iota16 + cc * L for cc in range(C // L)]

            def pbody(p, _, g=g, b=b, bw=bw, lb=lb, ccoff=ccoff):
                ps = jnp.full((L,), 0, jnp.int32) + p
                x0, x1, y0, y1, z0, z1 = [jnp.take(w, ps) for w in bw]
                lbb = [jnp.take(r, ps) for r in lb]
                wb = [y0 * z0, y0 * z1, y1 * z0, y1 * z1,
                      x0 * z0, x0 * z1, x1 * z0, x1 * z1,
                      x0 * y0, x0 * y1, x1 * y0, x1 * y1]
                lwb = [x0, x1, y0, y1, z0, z1]
                q = g * L + p
                pdst = q * PRODP
                chunks = []
                for i in range(3):
                    for cc in range(C // L):
                        acc = None
                        for t in range(4):
                            v = tap[b][(4 * i + t) * K + q, pl.ds(cc * L, L)]
                            term = v * wb[4 * i + t]
                            acc = term if acc is None else acc + term
                        la = lbb[i] + ccoff[cc]
                        l0 = plsc.load_gather(ltab_v, [la])
                        l1 = plsc.load_gather(ltab_v, [la + C])
                        lv = l0 * lwb[2 * i] + l1 * lwb[2 * i + 1]
                        chunks.append(acc * lv)
                chunks.append(jnp.zeros((L,), jnp.float32))
                for pr in range(PRODP // (2 * L)):
                    packed = plsc.pack(chunks[2 * pr], chunks[2 * pr + 1],
                                       format=plsc.PackFormat.INTERLEAVED)
                    prodb[b][pl.ds(pdst + pr * 2 * L, 2 * L)] = packed
                return _

            plsc.parallel_loop(0, L, 1, unroll=2, carry=jnp.int32(0))(pbody)

    pltpu.sync_copy(xs.at[pl.ds(wbase, K)], xv[0])
    pltpu.sync_copy(ys.at[pl.ds(wbase, K)], yv[0])
    pltpu.sync_copy(zs.at[pl.ds(wbase, K)], zv[0])
    pass1(jnp.int32(0), 0)
    issue_gathers(0)
    issue_xyz(jnp.int32(1), 1)

    def pair_body(i, carry):
        for b in (0, 1):
            c = 2 * i + b
            b2 = 1 - b
            wait_gathers(b)
            wait_xyz(b2)
            cn = jnp.minimum(c + 1, N_CHUNKS - 1)
            pass1(cn, b2)
            issue_gathers(b2)
            issue_xyz(jnp.minimum(c + 2, N_CHUNKS - 1), b)

            @pl.when(i >= 1)
            def _wait_prod(b=b):
                pltpu.make_async_copy(prodb[b],
                                      prod_out.at[pl.ds(0, K * PRODP)],
                                      psem[b]).wait()

            combine(c, b)
            pltpu.async_copy(prodb[b],
                             prod_out.at[pl.ds((wbase + c * K) * PRODP,
                                               K * PRODP)], psem[b])
        return carry

    lax.fori_loop(0, N_PAIRS, pair_body, 0, unroll=False)

    wait_gathers(0)
    wait_xyz(1)
    for b in (0, 1):
        pltpu.make_async_copy(prodb[b], prod_out.at[pl.ds(0, K * PRODP)],
                              psem[b]).wait()


def _mm_body(prod_ref, wt_ref, out_ref):
    out_ref[...] = jnp.dot(prod_ref[...], wt_ref[...],
                           preferred_element_type=jnp.float32)


def kernel(xyz_sampled, plane0, plane1, plane2, line0, line1, line2, W_basis):
    ptab = jnp.stack([plane0, plane1, plane2])
    ptab = ptab.transpose(0, 2, 3, 1).reshape(3 * RES * RES, C)
    ltab = jnp.stack([line0, line1, line2])
    ltab = ltab.reshape(3, C, RES).transpose(0, 2, 1).reshape(3 * RES * C)
    xyz = jnp.pad(xyz_sampled, ((0, N_PAD - N_POINTS), (0, 0)))
    xs = xyz[:, 0]
    ys = xyz[:, 1]
    zs = xyz[:, 2]

    mesh = plsc.VectorSubcoreMesh(core_axis_name="c", subcore_axis_name="s",
                                  num_cores=NC, num_subcores=NS)
    sc = pl.kernel(
        _sc_body,
        out_type=jax.ShapeDtypeStruct((N_PAD * PRODP,), jnp.bfloat16),
        mesh=mesh,
        compiler_params=pltpu.CompilerParams(
            needs_layout_passes=False, use_tc_tiling_on_sc=False),
        scratch_types=[
            pltpu.VMEM((K,), jnp.float32),
            pltpu.VMEM((K,), jnp.float32),
            pltpu.VMEM((K,), jnp.float32),
            pltpu.VMEM((K,), jnp.float32),
            pltpu.VMEM((K,), jnp.float32),
            pltpu.VMEM((K,), jnp.float32),
            pltpu.VMEM((3 * RES * C,), jnp.float32),
            pltpu.VMEM((12, K), jnp.int32),
            pltpu.VMEM((12, K), jnp.int32),
            pltpu.VMEM((6, K), jnp.float32),
            pltpu.VMEM((6, K), jnp.float32),
            pltpu.VMEM((3, K), jnp.int32),
            pltpu.VMEM((3, K), jnp.int32),
            pltpu.VMEM((12 * K, C), jnp.float32),
            pltpu.VMEM((12 * K, C), jnp.float32),
            pltpu.VMEM((K * PRODP,), jnp.bfloat16),
            pltpu.VMEM((K * PRODP,), jnp.bfloat16),
            pltpu.SemaphoreType.DMA,
            pltpu.SemaphoreType.DMA,
            pltpu.SemaphoreType.DMA,
            pltpu.SemaphoreType.DMA,
            pltpu.SemaphoreType.DMA,
            pltpu.SemaphoreType.DMA,
        ],
    )
    prod = sc(xs, ys, zs, ptab, ltab).reshape(N_PAD, PRODP)

    wt = jnp.pad(W_basis.T, ((0, PRODP - PROD), (0, 0)))
    wt = wt.reshape(PRODP // 32, 2, L, F).transpose(0, 2, 1, 3)
    wt = wt.reshape(PRODP, F).astype(jnp.bfloat16)
    BN = 1024
    grid = (N_PAD // BN,)
    out = pl.pallas_call(
        _mm_body,
        grid=grid,
        in_specs=[
            pl.BlockSpec((BN, PRODP), lambda i: (i, 0)),
            pl.BlockSpec((PRODP, F), lambda i: (0, 0)),
        ],
        out_specs=pl.BlockSpec((BN, F), lambda i: (i, 0)),
        out_shape=jax.ShapeDtypeStruct((N_POINTS, F), jnp.float32),
    )(prod, wt)
    return out


# speedup vs baseline: 1.3122x; 1.3122x over previous
import jax
import jax.numpy as jnp
from jax import lax
from jax.experimental import pallas as pl
from jax.experimental.pallas import tpu as pltpu
from jax.experimental.pallas import tpu_sc as plsc

N_POINTS = 500000
C = 48
RES = 300
F = 27
NC, NS, L = 2, 16, 16
NW = NC * NS
K = 48
N_CHUNKS = 326
N_PAD = NW * K * N_CHUNKS
PER_W = N_PAD // NW
N_PAIRS = N_CHUNKS // 2
GROUPS = K // L
PROD = 3 * C
PRODP = 160


def _coord(v):
    t = (v + 1.0) * 0.5 * (RES - 1)
    fi = jnp.clip(t.astype(jnp.int32), 0, RES - 2)
    w1 = t - fi.astype(jnp.float32)
    return fi, 1.0 - w1, w1


def _sc_body(xs, ys, zs, ptab, ltab_hbm, prod_out,
             xv0, yv0, zv0, xv1, yv1, zv1, ltab_v,
             idx0, idx1, wb0, wb1, lb0, lb1, tap0, tap1, pr0, pr1,
             gsem0, gsem1, xsem0, xsem1, psem0, psem1):
    xv = (xv0, xv1)
    yv = (yv0, yv1)
    zv = (zv0, zv1)
    idxb = (idx0, idx1)
    wbuf = (wb0, wb1)
    lbuf = (lb0, lb1)
    tap = (tap0, tap1)
    prodb = (pr0, pr1)
    gsem = (gsem0, gsem1)
    xsem = (xsem0, xsem1)
    psem = (psem0, psem1)

    wid = lax.axis_index("s") * NC + lax.axis_index("c")
    wbase = wid * PER_W
    iota16 = lax.iota(jnp.int32, L)

    pltpu.sync_copy(ltab_hbm, ltab_v)

    def pass1(cn, b):
        base = wbase + cn * K
        for g in range(GROUPS):
            sl = pl.ds(g * L, L)
            xi, xw0, xw1 = _coord(xv[b][sl])
            yi, yw0, yw1 = _coord(yv[b][sl])
            zi, zw0, zw1 = _coord(zv[b][sl])
            r0 = yi * RES + zi
            r1 = xi * RES + zi + RES * RES
            r2 = xi * RES + yi + 2 * RES * RES
            for t, r in enumerate((r0, r1, r2)):
                idxb[b][4 * t + 0, sl] = r
                idxb[b][4 * t + 1, sl] = r + 1
                idxb[b][4 * t + 2, sl] = r + RES
                idxb[b][4 * t + 3, sl] = r + RES + 1
            wbuf[b][0, sl] = xw0
            wbuf[b][1, sl] = xw1
            wbuf[b][2, sl] = yw0
            wbuf[b][3, sl] = yw1
            wbuf[b][4, sl] = zw0
            wbuf[b][5, sl] = zw1
            lbuf[b][0, sl] = xi * C
            lbuf[b][1, sl] = yi * C + RES * C
            lbuf[b][2, sl] = zi * C + 2 * RES * C

    def issue_gathers(b):
        for t in range(12):
            pltpu.async_copy(ptab.at[idxb[b].at[t]],
                             tap[b].at[pl.ds(t * K, K)], gsem[b])

    def wait_gathers(b):
        pltpu.make_async_copy(ptab.at[pl.ds(0, 12 * K)], tap[b],
                              gsem[b]).wait()

    def issue_xyz(c, b):
        base = wbase + c * K
        pltpu.async_copy(xs.at[pl.ds(base, K)], xv[b], xsem[b])
        pltpu.async_copy(ys.at[pl.ds(base, K)], yv[b], xsem[b])
        pltpu.async_copy(zs.at[pl.ds(base, K)], zv[b], xsem[b])

    def wait_xyz(b):
        for r in (xv[b], yv[b], zv[b]):
            pltpu.make_async_copy(xs.at[pl.ds(0, K)], r, xsem[b]).wait()

    def combine(c, b):
        for g in range(GROUPS):
            sl = pl.ds(g * L, L)
            bw = [wbuf[b][j, sl] for j in range(6)]
            lb = [lbuf[b][j, sl] for j in range(3)]
            ccoff = [iota16 + cc * L for cc in range(C // L)]

            def pbody(p, _, g=g, b=b, bw=bw, lb=lb, ccoff=ccoff):
                ps = jnp.full((L,), 0, jnp.int32) + p
                x0, x1, y0, y1, z0, z1 = [jnp.take(w, ps) for w in bw]
                lbb = [jnp.take(r, ps) for r in lb]
                wb = [y0 * z0, y0 * z1, y1 * z0, y1 * z1,
                      x0 * z0, x0 * z1, x1 * z0, x1 * z1,
                      x0 * y0, x0 * y1, x1 * y0, x1 * y1]
                lwb = [x0, x1, y0, y1, z0, z1]
                q = g * L + p
                pdst = q * PROD
                for i in range(3):
                    for cc in range(C // L):
                        acc = None
                        for t in range(4):
                            v = tap[b][(4 * i + t) * K + q, pl.ds(cc * L, L)]
                            term = v * wb[4 * i + t]
                            acc = term if acc is None else acc + term
                        la = lbb[i] + ccoff[cc]
                        l0 = plsc.load_gather(ltab_v, [la])
                        l1 = plsc.load_gather(ltab_v, [la + C])
                        lv = l0 * lwb[2 * i] + l1 * lwb[2 * i + 1]
                        prodb[b][pl.ds(pdst + i * C + cc * L, L)] = acc * lv
                return _

            plsc.parallel_loop(0, L, 1, unroll=2, carry=jnp.int32(0))(pbody)

    pltpu.sync_copy(xs.at[pl.ds(wbase, K)], xv[0])
    pltpu.sync_copy(ys.at[pl.ds(wbase, K)], yv[0])
    pltpu.sync_copy(zs.at[pl.ds(wbase, K)], zv[0])
    pass1(jnp.int32(0), 0)
    issue_gathers(0)
    issue_xyz(jnp.int32(1), 1)

    def pair_body(i, carry):
        for b in (0, 1):
            c = 2 * i + b
            b2 = 1 - b
            wait_gathers(b)
            wait_xyz(b2)
            cn = jnp.minimum(c + 1, N_CHUNKS - 1)
            pass1(cn, b2)
            issue_gathers(b2)
            issue_xyz(jnp.minimum(c + 2, N_CHUNKS - 1), b)

            @pl.when(i >= 1)
            def _wait_prod(b=b):
                pltpu.make_async_copy(prodb[b],
                                      prod_out.at[pl.ds(0, K * PROD)],
                                      psem[b]).wait()

            combine(c, b)
            pltpu.async_copy(prodb[b],
                             prod_out.at[pl.ds((wbase + c * K) * PROD,
                                               K * PROD)], psem[b])
        return carry

    lax.fori_loop(0, N_PAIRS, pair_body, 0, unroll=False)

    wait_gathers(0)
    wait_xyz(1)
    for b in (0, 1):
        pltpu.make_async_copy(prodb[b], prod_out.at[pl.ds(0, K * PROD)],
                              psem[b]).wait()


def _mm_body(prod_ref, wt_ref, out_ref):
    out_ref[...] = jnp.dot(prod_ref[...], wt_ref[...],
                           preferred_element_type=jnp.float32)


def kernel(xyz_sampled, plane0, plane1, plane2, line0, line1, line2, W_basis):
    ptab = jnp.stack([plane0, plane1, plane2])
    ptab = ptab.transpose(0, 2, 3, 1).reshape(3 * RES * RES, C)
    ltab = jnp.stack([line0, line1, line2])
    ltab = ltab.reshape(3, C, RES).transpose(0, 2, 1).reshape(3 * RES * C)
    xyz = jnp.pad(xyz_sampled, ((0, N_PAD - N_POINTS), (0, 0)))
    xs = xyz[:, 0]
    ys = xyz[:, 1]
    zs = xyz[:, 2]

    mesh = plsc.VectorSubcoreMesh(core_axis_name="c", subcore_axis_name="s",
                                  num_cores=NC, num_subcores=NS)
    sc = pl.kernel(
        _sc_body,
        out_type=jax.ShapeDtypeStruct((N_PAD * PROD,), jnp.float32),
        mesh=mesh,
        compiler_params=pltpu.CompilerParams(
            needs_layout_passes=False, use_tc_tiling_on_sc=False),
        scratch_types=[
            pltpu.VMEM((K,), jnp.float32),
            pltpu.VMEM((K,), jnp.float32),
            pltpu.VMEM((K,), jnp.float32),
            pltpu.VMEM((K,), jnp.float32),
            pltpu.VMEM((K,), jnp.float32),
            pltpu.VMEM((K,), jnp.float32),
            pltpu.VMEM((3 * RES * C,), jnp.float32),
            pltpu.VMEM((12, K), jnp.int32),
            pltpu.VMEM((12, K), jnp.int32),
            pltpu.VMEM((6, K), jnp.float32),
            pltpu.VMEM((6, K), jnp.float32),
            pltpu.VMEM((3, K), jnp.int32),
            pltpu.VMEM((3, K), jnp.int32),
            pltpu.VMEM((12 * K, C), jnp.float32),
            pltpu.VMEM((12 * K, C), jnp.float32),
            pltpu.VMEM((K * PROD,), jnp.float32),
            pltpu.VMEM((K * PROD,), jnp.float32),
            pltpu.SemaphoreType.DMA,
            pltpu.SemaphoreType.DMA,
            pltpu.SemaphoreType.DMA,
            pltpu.SemaphoreType.DMA,
            pltpu.SemaphoreType.DMA,
            pltpu.SemaphoreType.DMA,
        ],
    )
    prod = sc(xs, ys, zs, ptab, ltab).reshape(N_PAD, PROD)

    BN = 1024
    grid = (N_PAD // BN,)
    out = pl.pallas_call(
        _mm_body,
        grid=grid,
        in_specs=[
            pl.BlockSpec((BN, PROD), lambda i: (i, 0)),
            pl.BlockSpec((PROD, F), lambda i: (0, 0)),
        ],
        out_specs=pl.BlockSpec((BN, F), lambda i: (i, 0)),
        out_shape=jax.ShapeDtypeStruct((N_POINTS, F), jnp.float32),
    )(prod, W_basis.T)
    return out
